# baseline (speedup 1.0000x reference)
import jax
import jax.numpy as jnp
from jax.experimental import pallas as pl


def _chamfer_body(a_ref, bt_ref, out_ref):
    a = a_ref[0]
    bt = bt_ref[0]
    col = jax.lax.broadcasted_iota(jnp.int32, a.shape, 1)
    a_aug = jnp.where(col == 3, 1.0, -2.0 * a)
    b2 = jnp.sum(bt * bt, axis=0, keepdims=True)
    row = jax.lax.broadcasted_iota(jnp.int32, bt.shape, 0)
    bt_aug = jnp.where(row == 3, b2, bt)
    d = jnp.dot(a_aug, bt_aug, preferred_element_type=jnp.float32)
    m = jnp.min(d, axis=1)
    a2 = jnp.sum(a * a, axis=1)
    loss = jnp.mean(a2 + m)
    out_ref[...] = jnp.broadcast_to(loss, out_ref.shape)


def kernel(adv_pc, ori_pc):
    B, K, _ = adv_pc.shape
    N = ori_pc.shape[1]
    a = jnp.pad(adv_pc, ((0, 0), (0, 0), (0, 5)))
    bt = jnp.pad(ori_pc, ((0, 0), (0, 0), (0, 5))).transpose(0, 2, 1)
    out = pl.pallas_call(
        _chamfer_body,
        grid=(B,),
        in_specs=[
            pl.BlockSpec((1, K, 8), lambda b: (b, 0, 0)),
            pl.BlockSpec((1, 8, N), lambda b: (b, 0, 0)),
        ],
        out_specs=pl.BlockSpec((1, 1, 128), lambda b: (b, 0, 0)),
        out_shape=jax.ShapeDtypeStruct((B, 1, 128), jnp.float32),
    )(a, bt)
    return jnp.mean(out[:, 0, 0])


# speedup vs baseline: 1.4050x; 1.4050x over previous
import jax
import jax.numpy as jnp
from jax.experimental import pallas as pl


def _chamfer_body(at_ref, bt_ref, out_ref):
    at = at_ref[0]
    bt = bt_ref[0]
    row_a = jax.lax.broadcasted_iota(jnp.int32, at.shape, 0)
    a_aug = jnp.where(row_a == 3, 1.0, -2.0 * at)
    b2 = jnp.sum(bt * bt, axis=0, keepdims=True)
    row_b = jax.lax.broadcasted_iota(jnp.int32, bt.shape, 0)
    bt_aug = jnp.where(row_b == 3, b2, bt)
    d = jax.lax.dot_general(
        a_aug, bt_aug, (((0,), (0,)), ((), ())),
        preferred_element_type=jnp.float32)
    m = jnp.min(d, axis=1)
    a2 = jnp.sum(at * at, axis=0)
    loss = jnp.mean(a2 + m)
    out_ref[...] = jnp.broadcast_to(loss, out_ref.shape)


def kernel(adv_pc, ori_pc):
    B, K, _ = adv_pc.shape
    N = ori_pc.shape[1]
    at = jnp.pad(adv_pc, ((0, 0), (0, 0), (0, 5))).transpose(0, 2, 1)
    bt = jnp.pad(ori_pc, ((0, 0), (0, 0), (0, 5))).transpose(0, 2, 1)
    out = pl.pallas_call(
        _chamfer_body,
        grid=(B,),
        in_specs=[
            pl.BlockSpec((1, 8, K), lambda b: (b, 0, 0)),
            pl.BlockSpec((1, 8, N), lambda b: (b, 0, 0)),
        ],
        out_specs=pl.BlockSpec((1, 1, 128), lambda b: (b, 0, 0)),
        out_shape=jax.ShapeDtypeStruct((B, 1, 128), jnp.float32),
    )(at, bt)
    return jnp.mean(out[:, 0, 0])
